# baseline (speedup 1.0000x reference)
import jax
import jax.numpy as jnp
from jax.experimental import pallas as pl
from jax.experimental.pallas import tpu as pltpu

_N = 4096
_D = 128
_P = 2
_K = _P * _D
_SKIP = 0.8
_TINY = 1e-12
_BM = 128


def _graph_kernel(nf_ref, w_ref, init_ref, raw_ref, adj_ref, y_ref, yt_ref):
    i = pl.program_id(0)

    @pl.when(i == 0)
    def _compute_features():
        nf = nf_ref[...]
        scale = 1.0 / jnp.sqrt(jnp.float32(_P))
        cols = []
        for p in range(_P):
            w = w_ref[p, :][None, :]
            ctx = nf * w
            nrm = jnp.sqrt(jnp.sum(ctx * ctx, axis=1, keepdims=True))
            cols.append(ctx / jnp.maximum(nrm, _TINY) * scale)
        y = jnp.concatenate(cols, axis=1).astype(jnp.bfloat16)
        y_ref[...] = y
        yt_ref[...] = y.T

    yrow = y_ref[pl.ds(i * _BM, _BM), :]
    att = jnp.dot(yrow, yt_ref[...], preferred_element_type=jnp.float32)
    raw = jnp.maximum(att, 0.0)
    row_sum = jnp.sum(raw, axis=1, keepdims=True)
    inv = 1.0 / jnp.maximum(row_sum, _TINY)
    raw_ref[...] = raw
    adj_ref[...] = _SKIP * init_ref[...] + (1.0 - _SKIP) * (raw * inv)


def kernel(node_features, init_adj, W):
    grid = (_N // _BM,)
    raw, adj = pl.pallas_call(
        _graph_kernel,
        grid=grid,
        in_specs=[
            pl.BlockSpec((_N, _D), lambda i: (0, 0)),
            pl.BlockSpec((_P, _D), lambda i: (0, 0)),
            pl.BlockSpec((_BM, _N), lambda i: (i, 0)),
        ],
        out_specs=(
            pl.BlockSpec((_BM, _N), lambda i: (i, 0)),
            pl.BlockSpec((_BM, _N), lambda i: (i, 0)),
        ),
        out_shape=(
            jax.ShapeDtypeStruct((_N, _N), jnp.float32),
            jax.ShapeDtypeStruct((_N, _N), jnp.float32),
        ),
        scratch_shapes=[
            pltpu.VMEM((_N, _K), jnp.bfloat16),
            pltpu.VMEM((_K, _N), jnp.bfloat16),
        ],
    )(node_features, W, init_adj)
    return (raw, adj)


# speedup vs baseline: 1.0442x; 1.0442x over previous
import jax
import jax.numpy as jnp
from jax.experimental import pallas as pl
from jax.experimental.pallas import tpu as pltpu

_N = 4096
_D = 128
_P = 2
_K = _P * _D
_SKIP = 0.8
_TINY = 1e-12
_BM = 512
_CHUNK = 512


def _graph_kernel(nf_ref, w_ref, init_ref, raw_ref, adj_ref, yt_ref):
    i = pl.program_id(0)

    @pl.when(i == 0)
    def _compute_features():
        scale = 1.0 / jnp.sqrt(jnp.float32(_P))

        def _feat_body(r, carry):
            slr = pl.ds(r * _BM, _BM)
            nf = nf_ref[slr, :]
            cols = []
            for p in range(_P):
                w = w_ref[p, :][None, :]
                ctx = nf * w
                nrm = jnp.sqrt(jnp.sum(ctx * ctx, axis=1, keepdims=True))
                cols.append(ctx / jnp.maximum(nrm, _TINY) * scale)
            y = jnp.concatenate(cols, axis=1).astype(jnp.bfloat16)
            yt_ref[:, slr] = y.T
            return carry

        jax.lax.fori_loop(0, _N // _BM, _feat_body, 0)

    ycols = yt_ref[:, pl.ds(i * _BM, _BM)]
    def _sim_body(c, rs):
        sl = pl.ds(c * _CHUNK, _CHUNK)
        att = jax.lax.dot_general(
            ycols, yt_ref[:, sl],
            dimension_numbers=(((0,), (0,)), ((), ())),
            preferred_element_type=jnp.float32,
        )
        raw = jnp.maximum(att, 0.0)
        raw_ref[:, sl] = raw
        return rs + jnp.sum(raw, axis=1, keepdims=True)

    row_sum = jax.lax.fori_loop(0, _N // _CHUNK, _sim_body,
                                jnp.zeros((_BM, 1), jnp.float32))
    inv = (1.0 - _SKIP) / jnp.maximum(row_sum, _TINY)

    def _blend_body(c, carry):
        sl = pl.ds(c * _CHUNK, _CHUNK)
        adj_ref[:, sl] = _SKIP * init_ref[:, sl] + raw_ref[:, sl] * inv
        return carry

    jax.lax.fori_loop(0, _N // _CHUNK, _blend_body, 0)


def kernel(node_features, init_adj, W):
    grid = (_N // _BM,)
    raw, adj = pl.pallas_call(
        _graph_kernel,
        grid=grid,
        in_specs=[
            pl.BlockSpec((_N, _D), lambda i: (0, 0)),
            pl.BlockSpec((_P, _D), lambda i: (0, 0)),
            pl.BlockSpec((_BM, _N), lambda i: (i, 0)),
        ],
        out_specs=(
            pl.BlockSpec((_BM, _N), lambda i: (i, 0)),
            pl.BlockSpec((_BM, _N), lambda i: (i, 0)),
        ),
        out_shape=(
            jax.ShapeDtypeStruct((_N, _N), jnp.float32),
            jax.ShapeDtypeStruct((_N, _N), jnp.float32),
        ),
        scratch_shapes=[
            pltpu.VMEM((_K, _N), jnp.bfloat16),
        ],
    )(node_features, W, init_adj)
    return (raw, adj)


# speedup vs baseline: 1.0748x; 1.0293x over previous
import jax
import jax.numpy as jnp
from jax.experimental import pallas as pl
from jax.experimental.pallas import tpu as pltpu

_N = 4096
_D = 128
_P = 2
_K = _P * _D
_SKIP = 0.8
_TINY = 1e-12
_BM = 256


def _graph_kernel(nf_ref, w_ref, init_ref, raw_ref, adj_ref, y_ref, yt_ref):
    i = pl.program_id(0)

    @pl.when(i == 0)
    def _compute_features():
        nf = nf_ref[...]
        scale = 1.0 / jnp.sqrt(jnp.float32(_P))
        cols = []
        for p in range(_P):
            w = w_ref[p, :][None, :]
            ctx = nf * w
            nrm = jnp.sqrt(jnp.sum(ctx * ctx, axis=1, keepdims=True))
            cols.append(ctx / jnp.maximum(nrm, _TINY) * scale)
        y = jnp.concatenate(cols, axis=1).astype(jnp.bfloat16)
        y_ref[...] = y
        yt_ref[...] = y.T

    @pl.when(i > 0)
    def _compute_strip():
        j = i - 1
        yrow = y_ref[pl.ds(j * _BM, _BM), :]
        att = jnp.dot(yrow, yt_ref[...], preferred_element_type=jnp.float32)
        raw = jnp.maximum(att, 0.0)
        row_sum = jnp.sum(raw, axis=1, keepdims=True)
        inv = (1.0 - _SKIP) / jnp.maximum(row_sum, _TINY)
        raw_ref[...] = raw
        adj_ref[...] = _SKIP * init_ref[...] + raw * inv


def _lag(i):
    return (jnp.maximum(i - 1, 0), 0)


def kernel(node_features, init_adj, W):
    grid = (_N // _BM + 1,)
    raw, adj = pl.pallas_call(
        _graph_kernel,
        grid=grid,
        in_specs=[
            pl.BlockSpec((_N, _D), lambda i: (0, 0)),
            pl.BlockSpec((_P, _D), lambda i: (0, 0)),
            pl.BlockSpec((_BM, _N), _lag),
        ],
        out_specs=(
            pl.BlockSpec((_BM, _N), _lag),
            pl.BlockSpec((_BM, _N), _lag),
        ),
        out_shape=(
            jax.ShapeDtypeStruct((_N, _N), jnp.float32),
            jax.ShapeDtypeStruct((_N, _N), jnp.float32),
        ),
        scratch_shapes=[
            pltpu.VMEM((_N, _K), jnp.bfloat16),
            pltpu.VMEM((_K, _N), jnp.bfloat16),
        ],
    )(node_features, W, init_adj)
    return (raw, adj)


# speedup vs baseline: 1.0829x; 1.0075x over previous
import jax
import jax.numpy as jnp
from jax.experimental import pallas as pl
from jax.experimental.pallas import tpu as pltpu

_N = 4096
_D = 128
_P = 2
_K = _P * _D
_SKIP = 0.8
_TINY = 1e-12
_BM = 256


def _graph_kernel(nf_ref, w_ref, init_ref, raw_ref, adj_ref, y_ref, yt_ref):
    i = pl.program_id(0)

    @pl.when(i == 0)
    def _compute_features():
        nf = nf_ref[...]
        scale = 1.0 / jnp.sqrt(jnp.float32(_P))
        cols = []
        for p in range(_P):
            w = w_ref[p, :][None, :]
            ctx = nf * w
            nrm = jnp.sqrt(jnp.sum(ctx * ctx, axis=1, keepdims=True))
            cols.append(ctx / jnp.maximum(nrm, _TINY) * scale)
        y = jnp.concatenate(cols, axis=1).astype(jnp.bfloat16)
        y_ref[...] = y
        yt_ref[...] = y.T

    yrow = y_ref[pl.ds(i * _BM, _BM), :]
    att = jnp.dot(yrow, yt_ref[...], preferred_element_type=jnp.float32)
    raw = jnp.maximum(att, 0.0)
    row_sum = jnp.sum(raw, axis=1, keepdims=True)
    inv = 1.0 / jnp.maximum(row_sum, _TINY)
    raw_ref[...] = raw
    adj_ref[...] = _SKIP * init_ref[...] + (1.0 - _SKIP) * (raw * inv)


def kernel(node_features, init_adj, W):
    grid = (_N // _BM,)
    raw, adj = pl.pallas_call(
        _graph_kernel,
        grid=grid,
        in_specs=[
            pl.BlockSpec((_N, _D), lambda i: (0, 0)),
            pl.BlockSpec((_P, _D), lambda i: (0, 0)),
            pl.BlockSpec((_BM, _N), lambda i: (i, 0)),
        ],
        out_specs=(
            pl.BlockSpec((_BM, _N), lambda i: (i, 0)),
            pl.BlockSpec((_BM, _N), lambda i: (i, 0)),
        ),
        out_shape=(
            jax.ShapeDtypeStruct((_N, _N), jnp.float32),
            jax.ShapeDtypeStruct((_N, _N), jnp.float32),
        ),
        scratch_shapes=[
            pltpu.VMEM((_N, _K), jnp.bfloat16),
            pltpu.VMEM((_K, _N), jnp.bfloat16),
        ],
    )(node_features, W, init_adj)
    return (raw, adj)


# speedup vs baseline: 1.0881x; 1.0048x over previous
import jax
import jax.numpy as jnp
from jax.experimental import pallas as pl
from jax.experimental.pallas import tpu as pltpu

_N = 4096
_D = 128
_P = 2
_K = _P * _D
_SKIP = 0.8
_TINY = 1e-12
_BM = 256


def _graph_kernel(nf_ref, w_ref, init_ref, raw_ref, adj_ref, y_ref, yt_ref):
    i = pl.program_id(0)

    @pl.when(i == 0)
    def _compute_features():
        nf = nf_ref[...]
        scale = 1.0 / jnp.sqrt(jnp.float32(_P))
        cols = []
        for p in range(_P):
            w = w_ref[p, :][None, :]
            ctx = nf * w
            ssq = jnp.sum(ctx * ctx, axis=1, keepdims=True)
            inv_nrm = jax.lax.rsqrt(jnp.maximum(ssq, _TINY * _TINY)) * scale
            cols.append(ctx * inv_nrm)
        y = jnp.concatenate(cols, axis=1).astype(jnp.bfloat16)
        y_ref[...] = y
        yt_ref[...] = y.T

    yrow = y_ref[pl.ds(i * _BM, _BM), :]
    att = jnp.dot(yrow, yt_ref[...], preferred_element_type=jnp.float32)
    raw = jnp.maximum(att, 0.0)
    row_sum = jnp.sum(raw, axis=1, keepdims=True)
    inv = (1.0 - _SKIP) / jnp.maximum(row_sum, _TINY)
    raw_ref[...] = raw
    adj_ref[...] = _SKIP * init_ref[...] + raw * inv


def kernel(node_features, init_adj, W):
    grid = (_N // _BM,)
    raw, adj = pl.pallas_call(
        _graph_kernel,
        grid=grid,
        in_specs=[
            pl.BlockSpec((_N, _D), lambda i: (0, 0)),
            pl.BlockSpec((_P, _D), lambda i: (0, 0)),
            pl.BlockSpec((_BM, _N), lambda i: (i, 0)),
        ],
        out_specs=(
            pl.BlockSpec((_BM, _N), lambda i: (i, 0)),
            pl.BlockSpec((_BM, _N), lambda i: (i, 0)),
        ),
        out_shape=(
            jax.ShapeDtypeStruct((_N, _N), jnp.float32),
            jax.ShapeDtypeStruct((_N, _N), jnp.float32),
        ),
        scratch_shapes=[
            pltpu.VMEM((_N, _K), jnp.bfloat16),
            pltpu.VMEM((_K, _N), jnp.bfloat16),
        ],
    )(node_features, W, init_adj)
    return (raw, adj)


# speedup vs baseline: 1.1114x; 1.0215x over previous
import jax
import jax.numpy as jnp
from jax.experimental import pallas as pl
from jax.experimental.pallas import tpu as pltpu

_N = 4096
_D = 128
_P = 2
_K = _P * _D
_SKIP = 0.8
_TINY = 1e-12
_BM = 512
_HM = 256


def _graph_kernel(nf_ref, w_ref, init_ref, raw_ref, adj_ref, yt_ref):
    i = pl.program_id(0)

    @pl.when(i == 0)
    def _compute_features():
        scale = 1.0 / jnp.sqrt(jnp.float32(_P))

        def _feat_body(r, carry):
            slr = pl.ds(r * _BM, _BM)
            nf = nf_ref[slr, :]
            cols = []
            for p in range(_P):
                w = w_ref[p, :][None, :]
                ctx = nf * w
                ssq = jnp.sum(ctx * ctx, axis=1, keepdims=True)
                inv_nrm = jax.lax.rsqrt(jnp.maximum(ssq, _TINY * _TINY)) * scale
                cols.append(ctx * inv_nrm)
            y = jnp.concatenate(cols, axis=1).astype(jnp.bfloat16)
            yt_ref[:, slr] = y.T
            return carry

        jax.lax.fori_loop(0, _N // _BM, _feat_body, 0)

    for h in range(_BM // _HM):
        ycols = yt_ref[:, pl.ds(i * _BM + h * _HM, _HM)]
        att = jax.lax.dot_general(
            ycols, yt_ref[...],
            dimension_numbers=(((0,), (0,)), ((), ())),
            preferred_element_type=jnp.float32,
        )
        raw = jnp.maximum(att, 0.0)
        row_sum = jnp.sum(raw, axis=1, keepdims=True)
        inv = (1.0 - _SKIP) / jnp.maximum(row_sum, _TINY)
        sl = pl.ds(h * _HM, _HM)
        raw_ref[sl, :] = raw
        adj_ref[sl, :] = _SKIP * init_ref[sl, :] + raw * inv


def kernel(node_features, init_adj, W):
    grid = (_N // _BM,)
    raw, adj = pl.pallas_call(
        _graph_kernel,
        grid=grid,
        in_specs=[
            pl.BlockSpec((_N, _D), lambda i: (0, 0)),
            pl.BlockSpec((_P, _D), lambda i: (0, 0)),
            pl.BlockSpec((_BM, _N), lambda i: (i, 0)),
        ],
        out_specs=(
            pl.BlockSpec((_BM, _N), lambda i: (i, 0)),
            pl.BlockSpec((_BM, _N), lambda i: (i, 0)),
        ),
        out_shape=(
            jax.ShapeDtypeStruct((_N, _N), jnp.float32),
            jax.ShapeDtypeStruct((_N, _N), jnp.float32),
        ),
        scratch_shapes=[
            pltpu.VMEM((_K, _N), jnp.bfloat16),
        ],
    )(node_features, W, init_adj)
    return (raw, adj)


# speedup vs baseline: 1.1246x; 1.0119x over previous
import jax
import jax.numpy as jnp
from jax.experimental import pallas as pl
from jax.experimental.pallas import tpu as pltpu

_N = 4096
_D = 128
_P = 2
_K = _P * _D
_SKIP = 0.8
_TINY = 1e-12
_BM = 512
_HM = 256
_NBUF = 3
_NBLK = _N // _HM


def _graph_kernel(nf_ref, w_ref, init_hbm, raw_ref, adj_ref, yt_ref,
                  init_buf, init_sem):
    i = pl.program_id(0)

    def _init_copy(blk):
        slot = jax.lax.rem(blk, _NBUF)
        return pltpu.make_async_copy(
            init_hbm.at[pl.ds(blk * _HM, _HM), :],
            init_buf.at[slot],
            init_sem.at[slot],
        )

    @pl.when(i == 0)
    def _prologue():
        for b in range(_NBUF):
            _init_copy(b).start()
        scale = 1.0 / jnp.sqrt(jnp.float32(_P))

        def _feat_body(r, carry):
            slr = pl.ds(r * _HM, _HM)
            nf = nf_ref[slr, :]
            cols = []
            for p in range(_P):
                w = w_ref[p, :][None, :]
                ctx = nf * w
                ssq = jnp.sum(ctx * ctx, axis=1, keepdims=True)
                inv_nrm = jax.lax.rsqrt(jnp.maximum(ssq, _TINY * _TINY)) * scale
                cols.append(ctx * inv_nrm)
            y = jnp.concatenate(cols, axis=1).astype(jnp.bfloat16)
            yt_ref[:, slr] = y.T
            return carry

        jax.lax.fori_loop(0, _N // _HM, _feat_body, 0)

    for h in range(_BM // _HM):
        blk = i * (_BM // _HM) + h
        _init_copy(blk).wait()
        init_half = init_buf.at[jax.lax.rem(blk, _NBUF)]

        ycols = yt_ref[:, pl.ds(i * _BM + h * _HM, _HM)]
        att = jax.lax.dot_general(
            ycols, yt_ref[...],
            dimension_numbers=(((0,), (0,)), ((), ())),
            preferred_element_type=jnp.float32,
        )
        raw = jnp.maximum(att, 0.0)
        row_sum = jnp.sum(raw, axis=1, keepdims=True)
        inv = (1.0 - _SKIP) / jnp.maximum(row_sum, _TINY)
        sl = pl.ds(h * _HM, _HM)
        raw_ref[sl, :] = raw
        adj_ref[sl, :] = _SKIP * init_half[...] + raw * inv

        @pl.when(blk + _NBUF < _NBLK)
        def _prefetch_next():
            _init_copy(blk + _NBUF).start()


def kernel(node_features, init_adj, W):
    grid = (_N // _BM,)
    raw, adj = pl.pallas_call(
        _graph_kernel,
        grid=grid,
        in_specs=[
            pl.BlockSpec((_N, _D), lambda i: (0, 0)),
            pl.BlockSpec((_P, _D), lambda i: (0, 0)),
            pl.BlockSpec(memory_space=pl.ANY),
        ],
        out_specs=(
            pl.BlockSpec((_BM, _N), lambda i: (i, 0)),
            pl.BlockSpec((_BM, _N), lambda i: (i, 0)),
        ),
        out_shape=(
            jax.ShapeDtypeStruct((_N, _N), jnp.float32),
            jax.ShapeDtypeStruct((_N, _N), jnp.float32),
        ),
        scratch_shapes=[
            pltpu.VMEM((_K, _N), jnp.bfloat16),
            pltpu.VMEM((_NBUF, _HM, _N), jnp.float32),
            pltpu.SemaphoreType.DMA((_NBUF,)),
        ],
    )(node_features, W, init_adj)
    return (raw, adj)


# speedup vs baseline: 1.1251x; 1.0004x over previous
import jax
import jax.numpy as jnp
from jax.experimental import pallas as pl
from jax.experimental.pallas import tpu as pltpu

_N = 4096
_D = 128
_P = 2
_K = _P * _D
_SKIP = 0.8
_TINY = 1e-12
_BM = 256
_NBUF = 3
_NBLK = _N // _BM


def _graph_kernel(nf_ref, w_ref, init_hbm, raw_hbm, adj_hbm, yt_ref,
                  init_buf, raw_buf, adj_buf, init_sem, raw_sem, adj_sem):
    i = pl.program_id(0)

    def _init_copy(blk):
        slot = jax.lax.rem(blk, _NBUF)
        return pltpu.make_async_copy(
            init_hbm.at[pl.ds(blk * _BM, _BM), :],
            init_buf.at[slot],
            init_sem.at[slot],
        )

    def _out_copy(buf, hbm, sem, blk):
        slot = jax.lax.rem(blk, _NBUF)
        return pltpu.make_async_copy(
            buf.at[slot],
            hbm.at[pl.ds(blk * _BM, _BM), :],
            sem.at[slot],
        )

    @pl.when(i == 0)
    def _prologue():
        for b in range(_NBUF):
            _init_copy(b).start()
        scale = 1.0 / jnp.sqrt(jnp.float32(_P))

        def _feat_body(r, carry):
            slr = pl.ds(r * _BM, _BM)
            nf = nf_ref[slr, :]
            cols = []
            for p in range(_P):
                w = w_ref[p, :][None, :]
                ctx = nf * w
                ssq = jnp.sum(ctx * ctx, axis=1, keepdims=True)
                inv_nrm = jax.lax.rsqrt(jnp.maximum(ssq, _TINY * _TINY)) * scale
                cols.append(ctx * inv_nrm)
            y = jnp.concatenate(cols, axis=1).astype(jnp.bfloat16)
            yt_ref[:, slr] = y.T
            return carry

        jax.lax.fori_loop(0, _N // _BM, _feat_body, 0)

    ycols = yt_ref[:, pl.ds(i * _BM, _BM)]
    att = jax.lax.dot_general(
        ycols, yt_ref[...],
        dimension_numbers=(((0,), (0,)), ((), ())),
        preferred_element_type=jnp.float32,
    )
    raw = jnp.maximum(att, 0.0)
    row_sum = jnp.sum(raw, axis=1, keepdims=True)
    inv = (1.0 - _SKIP) / jnp.maximum(row_sum, _TINY)

    @pl.when(i >= _NBUF)
    def _wait_slot_drained():
        _out_copy(raw_buf, raw_hbm, raw_sem, i - _NBUF).wait()
        _out_copy(adj_buf, adj_hbm, adj_sem, i - _NBUF).wait()

    _init_copy(i).wait()
    slot = jax.lax.rem(i, _NBUF)
    raw_buf[slot] = raw
    adj_buf[slot] = _SKIP * init_buf[slot] + raw * inv
    _out_copy(raw_buf, raw_hbm, raw_sem, i).start()
    _out_copy(adj_buf, adj_hbm, adj_sem, i).start()

    @pl.when(i + _NBUF < _NBLK)
    def _prefetch_next():
        _init_copy(i + _NBUF).start()

    @pl.when(i == _NBLK - 1)
    def _drain_tail():
        for blk in range(_NBLK - _NBUF, _NBLK):
            _out_copy(raw_buf, raw_hbm, raw_sem, blk).wait()
            _out_copy(adj_buf, adj_hbm, adj_sem, blk).wait()


def kernel(node_features, init_adj, W):
    grid = (_NBLK,)
    raw, adj = pl.pallas_call(
        _graph_kernel,
        grid=grid,
        in_specs=[
            pl.BlockSpec((_N, _D), lambda i: (0, 0)),
            pl.BlockSpec((_P, _D), lambda i: (0, 0)),
            pl.BlockSpec(memory_space=pl.ANY),
        ],
        out_specs=(
            pl.BlockSpec(memory_space=pl.ANY),
            pl.BlockSpec(memory_space=pl.ANY),
        ),
        out_shape=(
            jax.ShapeDtypeStruct((_N, _N), jnp.float32),
            jax.ShapeDtypeStruct((_N, _N), jnp.float32),
        ),
        scratch_shapes=[
            pltpu.VMEM((_K, _N), jnp.bfloat16),
            pltpu.VMEM((_NBUF, _BM, _N), jnp.float32),
            pltpu.VMEM((_NBUF, _BM, _N), jnp.float32),
            pltpu.VMEM((_NBUF, _BM, _N), jnp.float32),
            pltpu.SemaphoreType.DMA((_NBUF,)),
            pltpu.SemaphoreType.DMA((_NBUF,)),
            pltpu.SemaphoreType.DMA((_NBUF,)),
        ],
    )(node_features, W, init_adj)
    return (raw, adj)


# speedup vs baseline: 1.1286x; 1.0031x over previous
import jax
import jax.numpy as jnp
from jax.experimental import pallas as pl
from jax.experimental.pallas import tpu as pltpu

_N = 4096
_D = 128
_P = 2
_K = _P * _D
_SKIP = 0.8
_TINY = 1e-12
_BM = 512
_HM = 256
_NBUF = 3
_NBLK = _N // _HM


def _graph_kernel(nf_ref, w_ref, init_hbm, raw_ref, adj_ref, yt_ref,
                  init_buf, init_sem):
    i = pl.program_id(0)

    def _init_copy(blk):
        slot = jax.lax.rem(blk, _NBUF)
        return pltpu.make_async_copy(
            init_hbm.at[pl.ds(blk * _HM, _HM), :],
            init_buf.at[slot],
            init_sem.at[slot],
        )

    @pl.when(i == 0)
    def _prologue():
        for b in range(_NBUF):
            _init_copy(b).start()
        scale = 1.0 / jnp.sqrt(jnp.float32(_P))

        def _feat_body(r, carry):
            slr = pl.ds(r * _HM, _HM)
            nf = nf_ref[slr, :]
            cols = []
            for p in range(_P):
                w = w_ref[p, :][None, :]
                ctx = nf * w
                ssq = jnp.sum(ctx * ctx, axis=1, keepdims=True)
                inv_nrm = jax.lax.rsqrt(jnp.maximum(ssq, _TINY * _TINY)) * scale
                cols.append(ctx * inv_nrm)
            y = jnp.concatenate(cols, axis=1).astype(jnp.bfloat16)
            yt_ref[:, slr] = y.T
            return carry

        jax.lax.fori_loop(0, _N // _HM, _feat_body, 0)

    for h in range(_BM // _HM):
        blk = i * (_BM // _HM) + h
        _init_copy(blk).wait()
        init_half = init_buf.at[jax.lax.rem(blk, _NBUF)]

        ycols = yt_ref[:, pl.ds(i * _BM + h * _HM, _HM)]
        att = jax.lax.dot_general(
            ycols, yt_ref[...],
            dimension_numbers=(((0,), (0,)), ((), ())),
            preferred_element_type=jnp.float32,
        )
        raw = jnp.maximum(att, 0.0)
        row_sum = jnp.sum(raw, axis=1, keepdims=True)
        inv = (1.0 - _SKIP) / jnp.maximum(row_sum, _TINY)
        sl = pl.ds(h * _HM, _HM)
        raw_ref[sl, :] = raw
        adj_ref[sl, :] = _SKIP * init_half[...] + raw * inv

        @pl.when(blk + _NBUF < _NBLK)
        def _prefetch_next():
            _init_copy(blk + _NBUF).start()


def kernel(node_features, init_adj, W):
    grid = (_N // _BM,)
    raw, adj = pl.pallas_call(
        _graph_kernel,
        grid=grid,
        in_specs=[
            pl.BlockSpec((_N, _D), lambda i: (0, 0)),
            pl.BlockSpec((_P, _D), lambda i: (0, 0)),
            pl.BlockSpec(memory_space=pl.ANY),
        ],
        out_specs=(
            pl.BlockSpec((_BM, _N), lambda i: (i, 0)),
            pl.BlockSpec((_BM, _N), lambda i: (i, 0)),
        ),
        out_shape=(
            jax.ShapeDtypeStruct((_N, _N), jnp.float32),
            jax.ShapeDtypeStruct((_N, _N), jnp.float32),
        ),
        scratch_shapes=[
            pltpu.VMEM((_K, _N), jnp.bfloat16),
            pltpu.VMEM((_NBUF, _HM, _N), jnp.float32),
            pltpu.SemaphoreType.DMA((_NBUF,)),
        ],
    )(node_features, W, init_adj)
    return (raw, adj)
